# Initial kernel scaffold; baseline (speedup 1.0000x reference)
import jax
import jax.numpy as jnp
from jax.experimental import pallas as pl


def kernel(mask, feats, coors, indices, W1, b1, W2, b2):
    raise NotImplementedError("write your pallas kernel here")


# speedup vs baseline: 26.8290x; 26.8290x over previous
import functools

import jax
import jax.numpy as jnp
from jax import lax
from jax.experimental import pallas as pl
from jax.experimental.pallas import tpu as pltpu
from jax.experimental.pallas import tpu_sc as plsc

F32 = jnp.float32

B, C, H, W = 2, 128, 100, 100
N, K = 10000, 16
M = B * N

NC, NS = 2, 16
NW = NC * NS
GN = 8
RPG = GN * K
TOT_GROUPS = M // GN
GROUPS_PER_W = -(-TOT_GROUPS // NW)
LANES = 16

TC_BLK = 2000


def _tc1_body(x_ref, cp_ref, wf_ref, wr1_ref, wr2_ref, b_ref,
              z_ref, p1_ref, p2_ref):
    cp = cp_ref[...]
    p1 = jnp.dot(cp, wr1_ref[...], preferred_element_type=F32)
    p2 = jnp.dot(cp, wr2_ref[...], preferred_element_type=F32)
    z = jnp.dot(x_ref[...], wf_ref[...], preferred_element_type=F32)
    z_ref[...] = z + b_ref[...] + p1
    p1_ref[...] = p1
    p2_ref[...] = p2


def _tc1(x, cp, wf, wr1, wr2, b):
    grid = (M // TC_BLK,)
    blk = lambda i: (i, 0)
    zero = lambda i: (0, 0)
    return pl.pallas_call(
        _tc1_body,
        grid=grid,
        in_specs=[
            pl.BlockSpec((TC_BLK, C), blk),
            pl.BlockSpec((TC_BLK, 8), blk),
            pl.BlockSpec((C, C), zero),
            pl.BlockSpec((8, C), zero),
            pl.BlockSpec((8, C), zero),
            pl.BlockSpec((1, C), zero),
        ],
        out_specs=[
            pl.BlockSpec((TC_BLK, C), blk),
            pl.BlockSpec((TC_BLK, C), blk),
            pl.BlockSpec((TC_BLK, C), blk),
        ],
        out_shape=[
            jax.ShapeDtypeStruct((M, C), F32),
            jax.ShapeDtypeStruct((M, C), F32),
            jax.ShapeDtypeStruct((M, C), F32),
        ],
    )(x, cp, wf, wr1, wr2, b)


def _tc2_body(x_ref, wf_ref, b_ref, p2_ref, z_ref):
    z = jnp.dot(x_ref[...], wf_ref[...], preferred_element_type=F32)
    z_ref[...] = z + b_ref[...] + p2_ref[...]


def _tc2(x, wf, b, p2):
    grid = (M // TC_BLK,)
    blk = lambda i: (i, 0)
    zero = lambda i: (0, 0)
    return pl.pallas_call(
        _tc2_body,
        grid=grid,
        in_specs=[
            pl.BlockSpec((TC_BLK, C), blk),
            pl.BlockSpec((C, C), zero),
            pl.BlockSpec((1, C), zero),
            pl.BlockSpec((TC_BLK, C), blk),
        ],
        out_specs=pl.BlockSpec((TC_BLK, C), blk),
        out_shape=jax.ShapeDtypeStruct((M, C), F32),
    )(x, wf, b, p2)


def _sc_body(z_hbm, p_hbm, x_hbm, gidx_hbm, out_hbm,
             idx_v, rows_v, p_v, acc_v, sem):
    wid = lax.axis_index("s") * NC + lax.axis_index("c")

    def group(j, carry):
        gg = wid + j * NW

        @pl.when(gg < TOT_GROUPS)
        def _():
            nb = pl.multiple_of(gg * GN, GN)
            pltpu.sync_copy(gidx_hbm.at[gg], idx_v)
            gat = pltpu.async_copy(z_hbm.at[idx_v.at[0]], rows_v, sem)
            pltpu.sync_copy(p_hbm.at[pl.ds(nb, GN)], p_v)
            pltpu.sync_copy(x_hbm.at[pl.ds(nb, GN)], acc_v)
            gat.wait()

            def node(n, carry_n):
                for c in range(C // LANES):
                    sl = pl.ds(c * LANES, LANES)
                    pv = p_v[n, sl]
                    a = acc_v[n, sl]
                    for k in range(K):
                        a = a + jnp.maximum(rows_v[n * K + k, sl] - pv, 0.0)
                    acc_v[n, sl] = a
                return carry_n

            lax.fori_loop(0, GN, node, 0)
            pltpu.sync_copy(acc_v, out_hbm.at[pl.ds(nb, GN)])

        return carry

    lax.fori_loop(0, GROUPS_PER_W, group, 0)


_sc_layer = functools.partial(
    pl.kernel,
    mesh=plsc.VectorSubcoreMesh(core_axis_name="c", subcore_axis_name="s"),
    out_type=jax.ShapeDtypeStruct((M, C), F32),
    scratch_types=[
        pltpu.VMEM((1, RPG), jnp.int32),
        pltpu.VMEM((RPG, C), F32),
        pltpu.VMEM((GN, C), F32),
        pltpu.VMEM((GN, C), F32),
        pltpu.SemaphoreType.DMA,
    ],
)(_sc_body)


@jax.jit
def kernel(mask, feats, coors, indices, W1, b1, W2, b2):
    x0 = jnp.transpose(feats, (0, 2, 3, 1)).reshape(B, H * W, C)
    x0 = jnp.where(mask.reshape(B, -1)[..., None], x0, 0.0).reshape(M, C)

    cp = jnp.concatenate([coors, jnp.zeros((B, N, 5), F32)], axis=-1)
    cp = cp.reshape(M, 8)
    zpad = jnp.zeros((5, C), F32)
    wr1 = jnp.concatenate([W1[C:], zpad], axis=0)
    wr2 = jnp.concatenate([W2[C:], zpad], axis=0)

    gidx = indices.astype(jnp.int32) + (
        jnp.arange(B, dtype=jnp.int32) * N)[:, None, None]
    gidx = gidx.reshape(TOT_GROUPS, 1, RPG)

    z1, p1, p2 = _tc1(x0, cp, W1[:C], wr1, wr2, b1.reshape(1, C))
    out1 = _sc_layer(z1, p1, x0, gidx)
    z2 = _tc2(out1, W2[:C], b2.reshape(1, C), p2)
    out2 = _sc_layer(z2, p2, out1, gidx)
    return out2.reshape(B, N, C)
